# Initial kernel scaffold; baseline (speedup 1.0000x reference)
import functools

import jax
import jax.numpy as jnp
from jax import lax
from jax.experimental import pallas as pl
from jax.experimental.pallas import tpu as pltpu
from jax.experimental.pallas import tpu_sc as plsc

_NC = 2
_NS = 16
_NW = _NC * _NS
_CHUNK = 128


@functools.lru_cache(maxsize=None)
def _make_gather(n_idx, vocab, emb_d):
    assert n_idx % (_NW * _CHUNK) == 0
    n_chunks = n_idx // (_NW * _CHUNK)

    @functools.partial(
        pl.kernel,
        out_type=jax.ShapeDtypeStruct((n_idx, emb_d), jnp.float32),
        mesh=plsc.VectorSubcoreMesh(core_axis_name="c", subcore_axis_name="s"),
        scratch_types=[
            pltpu.VMEM((n_chunks, _CHUNK), jnp.int32),
            pltpu.VMEM((_CHUNK, emb_d), jnp.float32),
            pltpu.SemaphoreType.DMA,
        ],
    )
    def gather_fn(idx_hbm, table_hbm, out_hbm, idx_v, rows_v, sem):
        wid = lax.axis_index("s") * _NC + lax.axis_index("c")
        base_chunk = wid * n_chunks
        pltpu.sync_copy(idx_hbm.at[pl.ds(base_chunk, n_chunks)], idx_v)

        def body(j, carry):
            pltpu.async_copy(table_hbm.at[idx_v.at[j]], rows_v, sem).wait()
            pltpu.sync_copy(
                rows_v, out_hbm.at[pl.ds((base_chunk + j) * _CHUNK, _CHUNK)]
            )
            return carry

        lax.fori_loop(0, n_chunks, body, 0)

    return gather_fn


def _matmul_body(x_ref, w_ref, b_ref, o_ref):
    o_ref[...] = (
        lax.dot_general(
            x_ref[...], w_ref[...], (((1,), (1,)), ((), ())),
            preferred_element_type=jnp.float32,
        )
        + b_ref[...]
    )


_BM = 512


@functools.lru_cache(maxsize=None)
def _make_matmul(batch, k_dim, out_d):
    return pl.pallas_call(
        _matmul_body,
        grid=(batch // _BM,),
        in_specs=[
            pl.BlockSpec((_BM, k_dim), lambda i: (i, 0)),
            pl.BlockSpec((out_d, k_dim), lambda i: (0, 0)),
            pl.BlockSpec((1, out_d), lambda i: (0, 0)),
        ],
        out_specs=pl.BlockSpec((_BM, out_d), lambda i: (i, 0)),
        out_shape=jax.ShapeDtypeStruct((batch, out_d), jnp.float32),
    )


def kernel(sentence, emb, W, b):
    batch, qlen = sentence.shape
    vocab, emb_d = emb.shape
    out_d = W.shape[0]
    n_idx = batch * qlen

    idx = sentence.reshape(n_idx // _CHUNK, _CHUNK)
    gathered = _make_gather(n_idx, vocab, emb_d)(idx, emb)
    x = gathered.reshape(batch, qlen * emb_d)
    return _make_matmul(batch, qlen * emb_d, out_d)(x, W, b.reshape(1, out_d))


# speedup vs baseline: 3.0910x; 3.0910x over previous
import functools

import jax
import jax.numpy as jnp
from jax import lax
from jax.experimental import pallas as pl
from jax.experimental.pallas import tpu as pltpu
from jax.experimental.pallas import tpu_sc as plsc

_NC = 2
_NS = 16
_NW = _NC * _NS
_CHUNK = 128


@functools.lru_cache(maxsize=None)
def _make_gather(n_idx, vocab, emb_d):
    assert n_idx % (_NW * _CHUNK) == 0
    n_chunks = n_idx // (_NW * _CHUNK)

    group_k = 5
    n_groups = n_chunks // group_k
    assert n_chunks % group_k == 0
    grp_rows = group_k * _CHUNK

    @functools.partial(
        pl.kernel,
        out_type=jax.ShapeDtypeStruct((n_idx, emb_d), jnp.float32),
        mesh=plsc.VectorSubcoreMesh(core_axis_name="c", subcore_axis_name="s"),
        compiler_params=pltpu.CompilerParams(use_tc_tiling_on_sc=False),
        scratch_types=[
            pltpu.VMEM((n_chunks, _CHUNK), jnp.int32),
            pltpu.VMEM((2, grp_rows, emb_d), jnp.float32),
            pltpu.SemaphoreType.DMA,
            pltpu.SemaphoreType.DMA,
        ],
    )
    def gather_fn(idx_hbm, table_hbm, out_hbm, idx_v, rows_v, gsem, wsem):
        wid = lax.axis_index("s") * _NC + lax.axis_index("c")
        base_row = wid * n_chunks * _CHUNK
        pltpu.sync_copy(idx_hbm.at[wid], idx_v)

        writebacks = {}
        for g in range(n_groups):
            p = g % 2
            if g >= 2:
                writebacks.pop(g - 2).wait()
            gathers = [
                pltpu.async_copy(
                    table_hbm.at[idx_v.at[g * group_k + t]],
                    rows_v.at[p, pl.ds(t * _CHUNK, _CHUNK)],
                    gsem,
                )
                for t in range(group_k)
            ]
            for cp in gathers:
                cp.wait()
            writebacks[g] = pltpu.async_copy(
                rows_v.at[p],
                out_hbm.at[pl.ds(base_row + g * grp_rows, grp_rows)],
                wsem,
            )
        for g in sorted(writebacks):
            writebacks.pop(g).wait()

    return gather_fn


def _matmul_body(x_ref, w_ref, b_ref, o_ref):
    o_ref[...] = (
        lax.dot_general(
            x_ref[...], w_ref[...], (((1,), (1,)), ((), ())),
            preferred_element_type=jnp.float32,
        )
        + b_ref[...]
    )


_BM = 512


@functools.lru_cache(maxsize=None)
def _make_matmul(batch, k_dim, out_d):
    return pl.pallas_call(
        _matmul_body,
        grid=(batch // _BM,),
        in_specs=[
            pl.BlockSpec((_BM, k_dim), lambda i: (i, 0)),
            pl.BlockSpec((out_d, k_dim), lambda i: (0, 0)),
            pl.BlockSpec((1, out_d), lambda i: (0, 0)),
        ],
        out_specs=pl.BlockSpec((_BM, out_d), lambda i: (i, 0)),
        out_shape=jax.ShapeDtypeStruct((batch, out_d), jnp.float32),
    )


def kernel(sentence, emb, W, b):
    batch, qlen = sentence.shape
    vocab, emb_d = emb.shape
    out_d = W.shape[0]
    n_idx = batch * qlen

    idx = sentence.reshape(_NW, n_idx // (_NW * _CHUNK), _CHUNK)
    gathered = _make_gather(n_idx, vocab, emb_d)(idx, emb)
    x = gathered.reshape(batch, qlen * emb_d)
    return _make_matmul(batch, qlen * emb_d, out_d)(x, W, b.reshape(1, out_d))


# speedup vs baseline: 3.4400x; 1.1129x over previous
import functools

import jax
import jax.numpy as jnp
from jax import lax
from jax.experimental import pallas as pl
from jax.experimental.pallas import tpu as pltpu
from jax.experimental.pallas import tpu_sc as plsc

_NC = 2
_NS = 16
_NW = _NC * _NS
_CHUNK = 128


@functools.lru_cache(maxsize=None)
def _make_gather(n_idx, vocab, emb_d):
    assert n_idx % (_NW * _CHUNK) == 0
    n_chunks = n_idx // (_NW * _CHUNK)

    group_k = 5
    n_groups = n_chunks // group_k
    assert n_chunks % group_k == 0
    grp_rows = group_k * _CHUNK

    @functools.partial(
        pl.kernel,
        out_type=jax.ShapeDtypeStruct((n_idx, emb_d), jnp.float32),
        mesh=plsc.VectorSubcoreMesh(core_axis_name="c", subcore_axis_name="s"),
        compiler_params=pltpu.CompilerParams(use_tc_tiling_on_sc=False),
        scratch_types=[
            pltpu.VMEM((n_chunks, _CHUNK), jnp.int32),
            pltpu.VMEM((2, grp_rows, emb_d), jnp.float32),
            pltpu.SemaphoreType.DMA,
            pltpu.SemaphoreType.DMA,
        ],
    )
    def gather_fn(idx_hbm, table_hbm, out_hbm, idx_v, rows_v, gsem, wsem):
        wid = lax.axis_index("s") * _NC + lax.axis_index("c")
        base_row = wid * n_chunks * _CHUNK
        pltpu.sync_copy(idx_hbm.at[wid], idx_v)

        writebacks = {}
        for g in range(n_groups):
            p = g % 2
            if g >= 2:
                writebacks.pop(g - 2).wait()
            gathers = [
                pltpu.async_copy(
                    table_hbm.at[idx_v.at[g * group_k + t]],
                    rows_v.at[p, pl.ds(t * _CHUNK, _CHUNK)],
                    gsem,
                )
                for t in range(group_k)
            ]
            for cp in gathers:
                cp.wait()
            writebacks[g] = pltpu.async_copy(
                rows_v.at[p],
                out_hbm.at[pl.ds(base_row + g * grp_rows, grp_rows)],
                wsem,
            )
        for g in sorted(writebacks):
            writebacks.pop(g).wait()

    return gather_fn


def _matmul_body(x_ref, w_ref, b_ref, o_ref):
    o_ref[...] = (
        lax.dot_general(
            w_ref[...], x_ref[...], (((0,), (1,)), ((), ())),
            preferred_element_type=jnp.float32,
        )
        + b_ref[...]
    )


_BM = 512


@functools.lru_cache(maxsize=None)
def _make_matmul(batch, k_dim, out_d):
    return pl.pallas_call(
        _matmul_body,
        grid=(batch // _BM,),
        in_specs=[
            pl.BlockSpec((_BM, k_dim), lambda i: (i, 0)),
            pl.BlockSpec((k_dim, out_d), lambda i: (0, 0)),
            pl.BlockSpec((out_d, 1), lambda i: (0, 0)),
        ],
        out_specs=pl.BlockSpec((out_d, _BM), lambda i: (0, i)),
        out_shape=jax.ShapeDtypeStruct((out_d, batch), jnp.float32),
    )


def kernel(sentence, emb, W, b):
    batch, qlen = sentence.shape
    vocab, emb_d = emb.shape
    out_d = W.shape[0]
    n_idx = batch * qlen

    idx = sentence.reshape(_NW, n_idx // (_NW * _CHUNK), _CHUNK)
    gathered = _make_gather(n_idx, vocab, emb_d)(idx, emb)
    x = gathered.reshape(batch, qlen * emb_d)
    out_t = _make_matmul(batch, qlen * emb_d, out_d)(
        x, W.T, b.reshape(out_d, 1)
    )
    return out_t.T
